# baseline (speedup 1.0000x reference)
import functools

import jax
import jax.numpy as jnp
from jax import lax
from jax.experimental import pallas as pl
from jax.experimental.pallas import tpu as pltpu
from jax.experimental.pallas import tpu_sc as plsc

V = 100000
E = 128
H = 256
B = 1024
L = 50

_CH = 80


@functools.partial(jax.jit, static_argnums=(2, 3))
def _sc_gather(table, idx3, n_tokens, d):
    info = plsc.get_sparse_core_info()
    nw = info.num_cores * info.num_subcores
    n_per_w = n_tokens // nw
    n_ch = idx3.shape[1]
    tail = n_per_w - (n_ch - 1) * _CH
    mesh = plsc.VectorSubcoreMesh(core_axis_name="c", subcore_axis_name="s")

    @functools.partial(
        pl.kernel,
        mesh=mesh,
        out_type=jax.ShapeDtypeStruct((n_tokens, d), jnp.float32),
        scratch_types=[
            pltpu.VMEM((n_ch, _CH), jnp.int32),
            pltpu.VMEM((_CH, d), jnp.float32),
            pltpu.VMEM((_CH, d), jnp.float32),
            pltpu.SemaphoreType.DMA,
            pltpu.SemaphoreType.DMA,
        ],
    )
    def gather_k(table_hbm, idx_hbm, out_hbm, idx_v, rows0, rows1, s0, s1):
        wid = lax.axis_index("s") * info.num_cores + lax.axis_index("c")
        base = wid * n_per_w
        pltpu.sync_copy(idx_hbm.at[wid], idx_v)
        bufs = (rows0, rows1)
        sems = (s0, s1)

        def start(j):
            b = j % 2
            return pltpu.async_copy(table_hbm.at[idx_v.at[j]], bufs[b], sems[b])

        h = [None, None]
        h[0] = start(0)
        for j in range(n_ch):
            b = j % 2
            if j + 1 < n_ch:
                h[1 - b] = start(j + 1)
            h[b].wait()
            pltpu.sync_copy(bufs[b], out_hbm.at[pl.ds(base + j * _CH, _CH)])

    return gather_k(table, idx3)


_UN = 8
_NG = -(-L // _UN)


def _lstm_body(x_ref, wih_ref, whh_ref, b_ref, out_ref, hn_ref, cn_ref,
               h_scr, c_scr):
    gi = pl.program_id(0)

    @pl.when(gi == 0)
    def _init():
        h_scr[...] = jnp.zeros_like(h_scr)
        c_scr[...] = jnp.zeros_like(c_scr)

    def _sigmoid(z):
        return 0.5 * jnp.tanh(0.5 * z) + 0.5

    wih = wih_ref[...].astype(jnp.bfloat16)
    whh = whh_ref[...].astype(jnp.bfloat16)
    bias = b_ref[...]
    h = h_scr[...]
    c = c_scr[...]
    for k in range(_UN):
        x = x_ref[pl.ds(k * B, B), :].astype(jnp.bfloat16)
        gates = (
            jnp.dot(x, wih, preferred_element_type=jnp.float32)
            + jnp.dot(h.astype(jnp.bfloat16), whh, preferred_element_type=jnp.float32)
            + bias
        )
        i = _sigmoid(gates[:, 0:H])
        f = _sigmoid(gates[:, H : 2 * H])
        g = jnp.tanh(gates[:, 2 * H : 3 * H])
        o = _sigmoid(gates[:, 3 * H : 4 * H])
        c = f * c + i * g
        h = o * jnp.tanh(c)
        out_ref[k] = h
        if k == (L - 1) - (_NG - 1) * _UN:
            @pl.when(gi == _NG - 1)
            def _write(h=h, c=c):
                hn_ref[...] = h
                cn_ref[...] = c
    h_scr[...] = h
    c_scr[...] = c


def _lstm(xs_lm, wih_t, whh_t, bias):
    return pl.pallas_call(
        _lstm_body,
        grid=(_NG,),
        in_specs=[
            pl.BlockSpec((_UN * B, E), lambda gi: (gi, 0)),
            pl.BlockSpec((E, 4 * H), lambda gi: (0, 0)),
            pl.BlockSpec((H, 4 * H), lambda gi: (0, 0)),
            pl.BlockSpec((1, 4 * H), lambda gi: (0, 0)),
        ],
        out_specs=[
            pl.BlockSpec((_UN, B, H), lambda gi: (gi, 0, 0)),
            pl.BlockSpec((B, H), lambda gi: (0, 0)),
            pl.BlockSpec((B, H), lambda gi: (0, 0)),
        ],
        out_shape=[
            jax.ShapeDtypeStruct((L, B, H), jnp.float32),
            jax.ShapeDtypeStruct((B, H), jnp.float32),
            jax.ShapeDtypeStruct((B, H), jnp.float32),
        ],
        scratch_shapes=[
            pltpu.VMEM((B, H), jnp.float32),
            pltpu.VMEM((B, H), jnp.float32),
        ],
        compiler_params=pltpu.CompilerParams(
            dimension_semantics=("arbitrary",),
        ),
    )(xs_lm, wih_t, whh_t, bias)


def kernel(input, table, W_ih, W_hh, b_ih, b_hh):
    n = B * L
    info = plsc.get_sparse_core_info()
    nw = info.num_cores * info.num_subcores
    idx3 = input.astype(jnp.int32).T.reshape(nw, (n // nw) // _CH, _CH)
    emb = _sc_gather(table, idx3, n, E)
    wih_t = W_ih.T
    whh_t = W_hh.T
    bias = (b_ih + b_hh).reshape(1, 4 * H)
    out_lbh, hn, cn = _lstm(emb, wih_t, whh_t, bias)
    out = jnp.swapaxes(out_lbh, 0, 1)
    return (out, hn[None, :, :], cn[None, :, :])


# speedup vs baseline: 1.0643x; 1.0643x over previous
import functools

import jax
import jax.numpy as jnp
from jax import lax
from jax.experimental import pallas as pl
from jax.experimental.pallas import tpu as pltpu
from jax.experimental.pallas import tpu_sc as plsc

V = 100000
E = 128
H = 256
B = 1024
L = 50

_UN = 8
_LA = 8


@functools.partial(jax.jit, static_argnums=(2, 3, 4))
def _sc_gather(table, idx3, n_tokens, d, ch):
    info = plsc.get_sparse_core_info()
    nw = info.num_cores * info.num_subcores
    n_per_w = n_tokens // nw
    n_ch = idx3.shape[1]
    mesh = plsc.VectorSubcoreMesh(core_axis_name="c", subcore_axis_name="s")

    @functools.partial(
        pl.kernel,
        mesh=mesh,
        out_type=jax.ShapeDtypeStruct((n_tokens, d), jnp.float32),
        scratch_types=[
            pltpu.VMEM((n_ch, ch), jnp.int32),
            pltpu.VMEM((ch, d), jnp.float32),
            pltpu.VMEM((ch, d), jnp.float32),
            pltpu.SemaphoreType.DMA,
            pltpu.SemaphoreType.DMA,
        ],
    )
    def gather_k(table_hbm, idx_hbm, out_hbm, idx_v, rows0, rows1, s0, s1):
        wid = lax.axis_index("s") * info.num_cores + lax.axis_index("c")
        base = wid * n_per_w
        pltpu.sync_copy(idx_hbm.at[wid], idx_v)
        bufs = (rows0, rows1)
        sems = (s0, s1)

        def start(j):
            b = j % 2
            return pltpu.async_copy(table_hbm.at[idx_v.at[j]], bufs[b], sems[b])

        h = [None, None]
        h[0] = start(0)
        for j in range(n_ch):
            b = j % 2
            if j + 1 < n_ch:
                h[1 - b] = start(j + 1)
            h[b].wait()
            pltpu.sync_copy(bufs[b], out_hbm.at[pl.ds(base + j * ch, ch)])

    return gather_k(table, idx3)


def _seg_body_first(x_ref, wih_ref, whh_ref, b_ref, out_ref, hn_ref, cn_ref,
                    h_scr, c_scr, *, ng, last_k, blk0):
    _seg_steps(None, None, x_ref, wih_ref, whh_ref, b_ref, out_ref, hn_ref,
               cn_ref, h_scr, c_scr, ng=ng, last_k=last_k, first=True)


def _seg_body_next(alias_ref, x_ref, wih_ref, whh_ref, b_ref, h0_ref, c0_ref,
                   out_ref, hn_ref, cn_ref, h_scr, c_scr, *, ng, last_k, blk0):
    _seg_steps(h0_ref, c0_ref, x_ref, wih_ref, whh_ref, b_ref, out_ref, hn_ref,
               cn_ref, h_scr, c_scr, ng=ng, last_k=last_k, first=False)


def _seg_steps(h0_ref, c0_ref, x_ref, wih_ref, whh_ref, b_ref, out_ref, hn_ref,
               cn_ref, h_scr, c_scr, *, ng, last_k, first):
    gi = pl.program_id(0)

    @pl.when(gi == 0)
    def _init():
        if first:
            h_scr[...] = jnp.zeros_like(h_scr)
            c_scr[...] = jnp.zeros_like(c_scr)
        else:
            h_scr[...] = h0_ref[...]
            c_scr[...] = c0_ref[...]

    def _sigmoid(z):
        return 0.5 * jnp.tanh(0.5 * z) + 0.5

    wih = wih_ref[...]
    whh = whh_ref[...]
    bias = b_ref[...]
    h = h_scr[...]
    c = c_scr[...]
    for k in range(_UN):
        x = x_ref[pl.ds(k * B, B), :]
        gates = (
            jnp.dot(x, wih, preferred_element_type=jnp.float32)
            + jnp.dot(h, whh, preferred_element_type=jnp.float32)
            + bias
        )
        i = _sigmoid(gates[:, 0:H])
        f = _sigmoid(gates[:, H : 2 * H])
        g = jnp.tanh(gates[:, 2 * H : 3 * H])
        o = _sigmoid(gates[:, 3 * H : 4 * H])
        c = f * c + i * g
        h = o * jnp.tanh(c)
        out_ref[k] = h
        if k == last_k:
            @pl.when(gi == ng - 1)
            def _write(h=h, c=c):
                hn_ref[...] = h
                cn_ref[...] = c
    h_scr[...] = h
    c_scr[...] = c


def _lstm_seg(xs, wih_t, whh_t, bias, h0c0, out_prev, l0, l_len):
    ng = -(-l_len // _UN)
    last_k = (l_len - 1) - (ng - 1) * _UN
    blk0 = l0 // _UN
    first = out_prev is None

    out_shape = [
        jax.ShapeDtypeStruct((L, B, H), jnp.float32),
        jax.ShapeDtypeStruct((B, H), jnp.float32),
        jax.ShapeDtypeStruct((B, H), jnp.float32),
    ]
    common_in = [
        pl.BlockSpec((_UN * B, E), lambda gi: (gi, 0)),
        pl.BlockSpec((E, 4 * H), lambda gi: (0, 0)),
        pl.BlockSpec((H, 4 * H), lambda gi: (0, 0)),
        pl.BlockSpec((1, 4 * H), lambda gi: (0, 0)),
    ]
    out_specs = [
        pl.BlockSpec((_UN, B, H), lambda gi, blk0=blk0: (gi + blk0, 0, 0)),
        pl.BlockSpec((B, H), lambda gi: (0, 0)),
        pl.BlockSpec((B, H), lambda gi: (0, 0)),
    ]
    scratch = [
        pltpu.VMEM((B, H), jnp.float32),
        pltpu.VMEM((B, H), jnp.float32),
    ]
    if first:
        return pl.pallas_call(
            functools.partial(_seg_body_first, ng=ng, last_k=last_k, blk0=blk0),
            grid=(ng,),
            in_specs=common_in,
            out_specs=out_specs,
            out_shape=out_shape,
            scratch_shapes=scratch,
            compiler_params=pltpu.CompilerParams(
                dimension_semantics=("arbitrary",),
            ),
        )(xs, wih_t, whh_t, bias)
    h0, c0 = h0c0
    return pl.pallas_call(
        functools.partial(_seg_body_next, ng=ng, last_k=last_k, blk0=blk0),
        grid=(ng,),
        in_specs=[pl.BlockSpec(memory_space=pl.ANY)]
        + common_in
        + [
            pl.BlockSpec((B, H), lambda gi: (0, 0)),
            pl.BlockSpec((B, H), lambda gi: (0, 0)),
        ],
        out_specs=out_specs,
        out_shape=out_shape,
        scratch_shapes=scratch,
        input_output_aliases={0: 0},
        compiler_params=pltpu.CompilerParams(
            dimension_semantics=("arbitrary",),
        ),
    )(out_prev, xs, wih_t, whh_t, bias, h0, c0)


def kernel(input, table, W_ih, W_hh, b_ih, b_hh):
    n = B * L
    info = plsc.get_sparse_core_info()
    nw = info.num_cores * info.num_subcores
    idx_lm = input.astype(jnp.int32).T.reshape(n)
    n_a = _LA * B
    n_b = n - n_a
    ch_a = 64
    ch_b = 112
    idx_a = idx_lm[:n_a].reshape(nw, (n_a // nw) // ch_a, ch_a)
    idx_b = idx_lm[n_a:].reshape(nw, (n_b // nw) // ch_b, ch_b)
    emb_a = _sc_gather(table, idx_a, n_a, E, ch_a)
    emb_b = _sc_gather(table, idx_b, n_b, E, ch_b)
    wih_t = W_ih.T
    whh_t = W_hh.T
    bias = (b_ih + b_hh).reshape(1, 4 * H)
    out_a, h_a, c_a = _lstm_seg(emb_a, wih_t, whh_t, bias, None, None, 0, _LA)
    out_b, hn, cn = _lstm_seg(
        emb_b, wih_t, whh_t, bias, (h_a, c_a), out_a, _LA, L - _LA
    )
    out = jnp.swapaxes(out_b, 0, 1)
    return (out, hn[None, :, :], cn[None, :, :])
